# baseline (speedup 1.0000x reference)
import functools

import jax
import jax.numpy as jnp
from jax import lax
from jax.experimental import pallas as pl
from jax.experimental.pallas import tpu as pltpu
from jax.experimental.pallas import tpu_sc as plsc

_B = 4096
_F = 26
_D = 64
_V = 100000
_NP1 = _F + 1


def _make_sc_gather(rows, nw, n_grp, grp, chunk_grps):
  rpw = rows // nw
  chunk_rows = chunk_grps * grp
  n_chunks = n_grp // chunk_grps
  mesh = plsc.VectorSubcoreMesh(core_axis_name="c", subcore_axis_name="s")
  nc = mesh.num_cores

  @functools.partial(
      pl.kernel,
      out_type=jax.ShapeDtypeStruct((rows, _D), jnp.float32),
      mesh=mesh,
      scratch_types=[
          pltpu.VMEM((n_grp, grp), jnp.int32),
          pltpu.VMEM((chunk_rows, _D), jnp.float32),
          pltpu.SemaphoreType.DMA,
      ],
      compiler_params=pltpu.CompilerParams(use_tc_tiling_on_sc=False),
  )
  def sc_gather(table_hbm, idx_hbm, out_hbm, idx_v, rows_v, sem):
    wid = lax.axis_index("s") * nc + lax.axis_index("c")
    base = wid * rpw
    pltpu.sync_copy(idx_hbm.at[wid], idx_v)
    for c in range(n_chunks):
      copies = []
      for g in range(chunk_grps):
        copies.append(pltpu.async_copy(
            table_hbm.at[idx_v.at[c * chunk_grps + g]],
            rows_v.at[pl.ds(g * grp, grp)],
            sem,
        ))
      for h in copies:
        h.wait()
      pltpu.sync_copy(rows_v,
                      out_hbm.at[pl.ds(base + c * chunk_rows, chunk_rows)])

  return sc_gather


def _tc_body(x_ref, emb_ref,
             dW0_ref, db0_ref, dW1_ref, db1_ref, dW2_ref, db2_ref,
             oW0d_ref, wsym_ref, ob0_ref, oW1_ref, ob1_ref,
             oW2_ref, ob2_ref, oW3_ref, ob3_ref, out_ref, *, bb):
  x = x_ref[...]
  h = jnp.maximum(jnp.dot(x, dW0_ref[...],
                          preferred_element_type=jnp.float32) + db0_ref[...], 0.0)
  h = jnp.maximum(jnp.dot(h, dW1_ref[...],
                          preferred_element_type=jnp.float32) + db1_ref[...], 0.0)
  d = jnp.maximum(jnp.dot(h, dW2_ref[...],
                          preferred_element_type=jnp.float32) + db2_ref[...], 0.0)
  emb = emb_ref[...].reshape(bb, _F, _D)
  comb = jnp.concatenate([d[:, None, :], emb], axis=1)
  inter = lax.dot_general(comb, comb, (((2,), (2,)), ((0,), (0,))),
                          preferred_element_type=jnp.float32)
  z = (jnp.dot(d, oW0d_ref[...], preferred_element_type=jnp.float32)
       + jnp.dot(inter.reshape(bb, _NP1 * _NP1), wsym_ref[...],
                 preferred_element_type=jnp.float32)
       + ob0_ref[...])
  z = jnp.maximum(z, 0.0)
  z = jnp.maximum(jnp.dot(z, oW1_ref[...],
                          preferred_element_type=jnp.float32) + ob1_ref[...], 0.0)
  z = jnp.maximum(jnp.dot(z, oW2_ref[...],
                          preferred_element_type=jnp.float32) + ob2_ref[...], 0.0)
  out_ref[...] = (jnp.dot(z, oW3_ref[...],
                          preferred_element_type=jnp.float32) + ob3_ref[...])


def kernel(dense_features, sparse_indices, embed_table,
           dW0, db0, dW1, db1, dW2, db2,
           oW0, ob0, oW1, ob1, oW2, ob2, oW3, ob3):
  rows = _B * _F
  nw = 32
  grp = 128
  n_grp = rows // (nw * grp)
  feat_offsets = (jnp.arange(_F, dtype=sparse_indices.dtype) * _V)[None, :]
  flat_idx = (sparse_indices + feat_offsets).reshape(nw, n_grp, grp)

  sc_gather = _make_sc_gather(rows, nw, n_grp, grp, chunk_grps=13)
  emb_rows = sc_gather(embed_table, flat_idx)
  emb = emb_rows.reshape(_B, _F * _D)

  xpad = jnp.pad(dense_features, ((0, 0), (0, 3)))
  dW0p = jnp.pad(dW0, ((0, 3), (0, 0)))

  oW0d = oW0[:_D]
  oW0p = oW0[_D:]
  ti, tj = jnp.triu_indices(_NP1, k=1)
  wsym = jnp.zeros((_NP1 * _NP1, 512), jnp.float32)
  wsym = wsym.at[ti * _NP1 + tj].set(0.5 * oW0p)
  wsym = wsym.at[tj * _NP1 + ti].set(0.5 * oW0p)

  bb = 256
  grid = _B // bb

  def first_dim_block(shape):
    return pl.BlockSpec((bb,) + shape[1:], lambda i: (i,) + (0,) * (len(shape) - 1))

  def whole(a):
    return pl.BlockSpec(a.shape, lambda i: (0,) * a.ndim)

  b2 = lambda v: v.reshape(1, -1)
  weights = (dW0p, b2(db0), dW1, b2(db1), dW2, b2(db2),
             oW0d, wsym, b2(ob0), oW1, b2(ob1), oW2, b2(ob2), oW3, b2(ob3))

  out = pl.pallas_call(
      functools.partial(_tc_body, bb=bb),
      grid=(grid,),
      in_specs=[first_dim_block(xpad.shape), first_dim_block(emb.shape)]
               + [whole(w) for w in weights],
      out_specs=first_dim_block((_B, 1)),
      out_shape=jax.ShapeDtypeStruct((_B, 1), jnp.float32),
  )(xpad, emb, *weights)
  return out


# speedup vs baseline: 1.5223x; 1.5223x over previous
import functools

import jax
import jax.numpy as jnp
from jax import lax
from jax.experimental import pallas as pl
from jax.experimental.pallas import tpu as pltpu
from jax.experimental.pallas import tpu_sc as plsc

_B = 4096
_F = 26
_D = 64
_V = 100000
_NP1 = _F + 1
_ROWS = _B * _F
_TROWS = _F * _V
_NBLK = (_TROWS + 127) // 128
_NW = 32
_G = 6
_LIST = 2 * _B + 48
_TRASH = _ROWS


def _make_sc_gather():
  mesh = plsc.VectorSubcoreMesh(core_axis_name="c", subcore_axis_name="s")
  nc = mesh.num_cores
  base_blk = _NBLK // _NW
  rem_blk = _NBLK % _NW

  @functools.partial(
      pl.kernel,
      out_type=jax.ShapeDtypeStruct((_ROWS + 16, 128), jnp.float32),
      mesh=mesh,
      scratch_types=[
          pltpu.VMEM((2, 32, 128), jnp.int32),
          pltpu.VMEM((_LIST,), jnp.int32),
          pltpu.VMEM((_LIST,), jnp.int32),
          pltpu.VMEM((_G, _D, 128), jnp.float32),
          pltpu.VMEM((_LIST,), jnp.int32),
          pltpu.VMEM((_LIST,), jnp.int32),
          pltpu.VMEM((1, 16), jnp.int32),
          pltpu.VMEM((16, 128), jnp.float32),
          pltpu.SemaphoreType.DMA,
          pltpu.SemaphoreType.DMA,
      ],
      compiler_params=pltpu.CompilerParams(use_tc_tiling_on_sc=True,
                                           needs_layout_passes=False),
  )
  def sc_gather(tableT, idxT3, tail_in, out_hbm, idx_v, list_r, list_p, buf,
                sub_r, sub_p, pos_st, dat_st, fsem, ssem):
    wid = lax.axis_index("s") * nc + lax.axis_index("c")
    w_blo = wid * base_blk + jnp.minimum(wid, rem_blk)
    nblk = base_blk + jnp.where(wid < rem_blk, 1, 0)
    r_lo = w_blo * 128
    r_hi = (w_blo + nblk) * 128
    iota = lax.iota(jnp.int32, 16)

    f0 = r_lo // _V
    f1 = jnp.minimum(f0 + 1, _F - 1)
    pltpu.sync_copy(idxT3.at[f0], idx_v.at[0])
    pltpu.sync_copy(idxT3.at[f1], idx_v.at[1])

    r_lo_v = jnp.full((16,), r_lo, jnp.int32)
    r_hi_v = jnp.full((16,), r_hi, jnp.int32)

    def scan_feature(fi, f, cur0):
      fv = jnp.full((16,), f, jnp.int32)

      def body(j, cur):
        rv = idx_v[fi, j // 8, pl.ds((j % 8) * 16, 16)]
        bv = jnp.full((16,), j * 16, jnp.int32) + iota
        pv = bv * _F + fv
        m = ((rv >= r_lo_v) & (rv < r_hi_v)).astype(jnp.int32)
        incl = plsc.cumsum(m)
        slot = jnp.where(m > 0, jnp.full((16,), cur, jnp.int32) + incl - m,
                         jnp.full((16,), _LIST - 1, jnp.int32))
        plsc.store_scatter(list_r, [slot], rv)
        plsc.store_scatter(list_p, [slot], pv)
        return cur + incl[15]

      return lax.fori_loop(0, _B // 16, body, cur0)

    cur = scan_feature(0, f0, jnp.int32(0))
    cur = scan_feature(1, f1, cur)
    list_r[pl.ds(cur, 16)] = jnp.full((16,), 2 ** 30, jnp.int32)
    list_p[pl.ds(cur, 16)] = jnp.full((16,), _TRASH, jnp.int32) + iota
    n_chunks = (cur + 15) // 16

    n_groups = (nblk + _G - 1) // _G

    def group_body(g, _):
      blk0 = w_blo + g * _G
      gn = jnp.minimum(nblk - g * _G, _G)
      has_tail = blk0 + gn == _NBLK
      full_n = gn - jnp.where(has_tail, 1, 0)

      def fetch(j, _):
        pltpu.async_copy(tableT.at[:, pl.ds((blk0 + j) * 128, 128)],
                         buf.at[j], fsem)
        return 0

      lax.fori_loop(0, full_n, fetch, 0)

      @pl.when(has_tail)
      def _():
        pltpu.sync_copy(tail_in, buf.at[full_n])

      blo_v = jnp.full((16,), blk0, jnp.int32)
      bhi_v = jnp.full((16,), blk0 + gn, jnp.int32)

      def rescan(j, scur):
        rv = list_r[pl.ds(j * 16, 16)]
        pv = list_p[pl.ds(j * 16, 16)]
        bv = lax.shift_right_logical(rv, 7)
        m = ((bv >= blo_v) & (bv < bhi_v)).astype(jnp.int32)
        incl = plsc.cumsum(m)
        slot = jnp.where(m > 0, jnp.full((16,), scur, jnp.int32) + incl - m,
                         jnp.full((16,), _LIST - 1, jnp.int32))
        plsc.store_scatter(sub_r, [slot], rv)
        plsc.store_scatter(sub_p, [slot], pv)
        return scur + incl[15]

      scur = lax.fori_loop(0, n_chunks, rescan, jnp.int32(0))
      sub_r[pl.ds(scur, 16)] = jnp.full((16,), blk0 * 128, jnp.int32)
      sub_p[pl.ds(scur, 16)] = jnp.full((16,), _TRASH, jnp.int32) + iota

      def drain(j, _):
        pltpu.make_async_copy(tableT.at[:, pl.ds(0, 128)], buf.at[j],
                              fsem).wait()
        return 0

      lax.fori_loop(0, full_n, drain, 0)

      nm = (scur + 15) // 16

      def ext_chunk(k, _):
        rv = sub_r[pl.ds(k * 16, 16)]
        pv = sub_p[pl.ds(k * 16, 16)]
        bv = lax.shift_right_logical(rv, 7)
        lane = rv - bv * 128
        blk_l = bv - blo_v
        pos_st[0] = pv
        for d in range(_D):
          vals = plsc.load_gather(
              buf, [blk_l, jnp.full((16,), d, jnp.int32), lane])
          plsc.store_scatter(dat_st, [iota, jnp.full((16,), d, jnp.int32)],
                             vals)
        pltpu.async_copy(dat_st, out_hbm.at[pos_st.at[0]], ssem).wait()
        return 0

      lax.fori_loop(0, nm, ext_chunk, 0)
      return 0

    lax.fori_loop(0, n_groups, group_body, 0)

  return sc_gather


def _tc_body(x_ref, emb_ref,
             dW0_ref, db0_ref, dW1_ref, db1_ref, dW2_ref, db2_ref,
             oW0d_ref, wsym_ref, ob0_ref, oW1_ref, ob1_ref,
             oW2_ref, ob2_ref, oW3_ref, ob3_ref, out_ref, *, bb):
  x = x_ref[...]
  h = jnp.maximum(jnp.dot(x, dW0_ref[...],
                          preferred_element_type=jnp.float32) + db0_ref[...], 0.0)
  h = jnp.maximum(jnp.dot(h, dW1_ref[...],
                          preferred_element_type=jnp.float32) + db1_ref[...], 0.0)
  d = jnp.maximum(jnp.dot(h, dW2_ref[...],
                          preferred_element_type=jnp.float32) + db2_ref[...], 0.0)
  emb = emb_ref[...].reshape(bb, _F, 128)[:, :, :_D]
  comb = jnp.concatenate([d[:, None, :], emb], axis=1)
  inter = lax.dot_general(comb, comb, (((2,), (2,)), ((0,), (0,))),
                          preferred_element_type=jnp.float32)
  z = (jnp.dot(d, oW0d_ref[...], preferred_element_type=jnp.float32)
       + jnp.dot(inter.reshape(bb, _NP1 * _NP1), wsym_ref[...],
                 preferred_element_type=jnp.float32)
       + ob0_ref[...])
  z = jnp.maximum(z, 0.0)
  z = jnp.maximum(jnp.dot(z, oW1_ref[...],
                          preferred_element_type=jnp.float32) + ob1_ref[...], 0.0)
  z = jnp.maximum(jnp.dot(z, oW2_ref[...],
                          preferred_element_type=jnp.float32) + ob2_ref[...], 0.0)
  out_ref[...] = (jnp.dot(z, oW3_ref[...],
                          preferred_element_type=jnp.float32) + ob3_ref[...])


def kernel(dense_features, sparse_indices, embed_table,
           dW0, db0, dW1, db1, dW2, db2,
           oW0, ob0, oW1, ob1, oW2, ob2, oW3, ob3):
  tableT = embed_table.T
  feat_offsets = (jnp.arange(_F, dtype=sparse_indices.dtype) * _V)[None, :]
  idxT3 = (sparse_indices + feat_offsets).T.astype(jnp.int32).reshape(
      _F, _B // 128, 128)
  tail_in = jnp.pad(embed_table[_TROWS - 64:].T, ((0, 0), (0, 64)))

  emb_rows = _make_sc_gather()(tableT, idxT3, tail_in)

  xpad = jnp.pad(dense_features, ((0, 0), (0, 3)))
  dW0p = jnp.pad(dW0, ((0, 3), (0, 0)))

  oW0d = oW0[:_D]
  oW0p = oW0[_D:]
  ti, tj = jnp.triu_indices(_NP1, k=1)
  wsym = jnp.zeros((_NP1 * _NP1, 512), jnp.float32)
  wsym = wsym.at[ti * _NP1 + tj].set(0.5 * oW0p)
  wsym = wsym.at[tj * _NP1 + ti].set(0.5 * oW0p)

  bb = 256
  grid = _B // bb

  def first_dim_block(shape):
    return pl.BlockSpec((bb,) + shape[1:], lambda i: (i,) + (0,) * (len(shape) - 1))

  def whole(a):
    return pl.BlockSpec(a.shape, lambda i: (0,) * a.ndim)

  b2 = lambda v: v.reshape(1, -1)
  weights = (dW0p, b2(db0), dW1, b2(db1), dW2, b2(db2),
             oW0d, wsym, b2(ob0), oW1, b2(ob1), oW2, b2(ob2), oW3, b2(ob3))

  out = pl.pallas_call(
      functools.partial(_tc_body, bb=bb),
      grid=(grid,),
      in_specs=[first_dim_block(xpad.shape),
                pl.BlockSpec((bb * _F, 128), lambda i: (i, 0))]
               + [whole(w) for w in weights],
      out_specs=first_dim_block((_B, 1)),
      out_shape=jax.ShapeDtypeStruct((_B, 1), jnp.float32),
  )(xpad, emb_rows, *weights)
  return out
